# baseline (speedup 1.0000x reference)
import functools

import jax
import jax.numpy as jnp
from jax import lax
from jax.experimental import pallas as pl
from jax.experimental.pallas import tpu as pltpu
from jax.experimental.pallas import tpu_sc as plsc

NC = 2
NS = 16
L = 16
NW = NC * NS


def _make_emb_mean(V, D, S, B):
  assert B % NW == 0 and D == 64 and V % 2 == 0 and S % 2 == 0
  b_per_w = B // NW
  assert b_per_w == 128
  W = 2 * D

  mesh = plsc.VectorSubcoreMesh(core_axis_name="c", subcore_axis_name="s")

  @functools.partial(
      pl.kernel,
      mesh=mesh,
      out_type=jax.ShapeDtypeStruct((B, D), jnp.float32),
      compiler_params=pltpu.CompilerParams(
          use_tc_tiling_on_sc=True, needs_layout_passes=False),
      scratch_types=[
          pltpu.VMEM((S, b_per_w), jnp.int32),
          pltpu.VMEM((S, b_per_w), jnp.int32),
          pltpu.VMEM((b_per_w, W), jnp.float32),
          pltpu.VMEM((b_per_w, W), jnp.float32),
          pltpu.VMEM((D, b_per_w), jnp.float32),
          pltpu.VMEM((b_per_w, D), jnp.float32),
          pltpu.SemaphoreType.DMA,
          pltpu.SemaphoreType.DMA,
      ],
  )
  def emb_mean(pairs_hbm, idx_hbm, out_hbm, qidx_v, pcol_v, rows0, rows1,
               accT, out_v, sem0, sem1):
    cid = lax.axis_index("c")
    sid = lax.axis_index("s")
    wid = cid * NS + sid
    base_glob = wid * b_per_w

    pltpu.sync_copy(idx_hbm.at[:, pl.ds(base_glob, b_per_w)], qidx_v)

    def split_body(s, carry):
      for c in range(b_per_w // L):
        v = qidx_v[s, pl.ds(c * L, L)]
        qidx_v[s, pl.ds(c * L, L)] = lax.shift_right_logical(v, 1)
        pcol_v[s, pl.ds(c * L, L)] = lax.shift_left(
            lax.bitwise_and(v, 1), 6)
      return carry

    lax.fori_loop(0, S, split_body, 0)

    zero = jnp.zeros((L,), jnp.float32)

    def zero_body(d, carry):
      for c in range(b_per_w // L):
        accT[d, pl.ds(c * L, L)] = zero
      return carry

    lax.fori_loop(0, D, zero_body, 0)

    bufs = (rows0, rows1)
    sems = (sem0, sem1)

    def start(s, p):
      pltpu.async_copy(pairs_hbm.at[qidx_v.at[s]], bufs[p], sems[p])

    def wait(p):
      pltpu.make_async_copy(
          pairs_hbm.at[qidx_v.at[0]], bufs[p], sems[p]).wait()

    lanes = lax.iota(jnp.int32, L)

    def accumulate(s, buf):
      for jc in range(b_per_w // L):
        rvec = lanes + (jc * L)
        col0 = pcol_v[s, pl.ds(jc * L, L)]

        def d_body(d8, col):
          for k in range(8):
            vals = plsc.load_gather(buf, [rvec, col])
            plsc.addupdate(accT.at[d8 * 8 + k, pl.ds(jc * L, L)], vals)
            col = col + 1
          return col

        lax.fori_loop(0, D // 8, d_body, col0)

    start(0, 0)
    start(1, 1)

    def body(k, carry):
      s = 2 * k
      for p in range(2):
        wait(p)
        accumulate(s + p, bufs[p])

        @pl.when(s + p + 2 < S)
        def _():
          start(s + p + 2, p)
      return carry

    lax.fori_loop(0, S // 2, body, 0)

    inv = jnp.float32(1.0 / S)

    def wb_body(j, carry):
      jsplat = jnp.full((L,), 0, jnp.int32) + j
      for k in range(D // L):
        vals = plsc.load_gather(accT, [lanes + (k * L), jsplat])
        out_v[j, pl.ds(k * L, L)] = vals * inv
      return carry

    lax.fori_loop(0, b_per_w, wb_body, 0)
    pltpu.sync_copy(out_v, out_hbm.at[pl.ds(base_glob, b_per_w)])

  return emb_mean


def kernel(embedding_weight, input_seqs):
  V, D = embedding_weight.shape
  S, B = input_seqs.shape
  idx = input_seqs.astype(jnp.int32)
  pairs = embedding_weight.reshape(V // 2, 2 * D)
  return _make_emb_mean(V, D, S, B)(pairs, idx)
